# baseline (speedup 1.0000x reference)
import functools

import jax
import jax.numpy as jnp
from jax import lax
from jax.experimental import pallas as pl
from jax.experimental.pallas import tpu as pltpu
from jax.experimental.pallas import tpu_sc as plsc

_B = 16384
_D = 32
_W = 5
_N = 5

_NC = 2
_NS = 16
_NW = _NC * _NS
_EPW = _B // _NW
_CHUNK = 128
_ROWS_PW = _EPW * _W
_NCH = _ROWS_PW // _CHUNK
_NCH_T = _EPW // _CHUNK


def _sc_gather_body(in_tab, out_tab, ctx_idx, tgt_idx, neg_idx,
                    ctx_out, tgt_out, neg_out, idx_v, rows_v, sem):
    c = lax.axis_index("c")
    s = lax.axis_index("s")
    wid = s * _NC + c

    pltpu.sync_copy(ctx_idx.at[wid], idx_v)
    cps = [pltpu.async_copy(in_tab.at[idx_v.at[j]],
                            rows_v.at[pl.ds(j * _CHUNK, _CHUNK)], sem)
           for j in range(_NCH)]
    for cp in cps:
        cp.wait()
    pltpu.sync_copy(rows_v, ctx_out.at[pl.ds(wid * _ROWS_PW, _ROWS_PW)])

    pltpu.sync_copy(neg_idx.at[wid], idx_v)
    cps = [pltpu.async_copy(out_tab.at[idx_v.at[j]],
                            rows_v.at[pl.ds(j * _CHUNK, _CHUNK)], sem)
           for j in range(_NCH)]
    for cp in cps:
        cp.wait()
    pltpu.sync_copy(rows_v, neg_out.at[pl.ds(wid * _ROWS_PW, _ROWS_PW)])

    pltpu.sync_copy(tgt_idx.at[wid], idx_v.at[pl.ds(0, _NCH_T)])
    cps = [pltpu.async_copy(out_tab.at[idx_v.at[j]],
                            rows_v.at[pl.ds(j * _CHUNK, _CHUNK)], sem)
           for j in range(_NCH_T)]
    for cp in cps:
        cp.wait()
    pltpu.sync_copy(rows_v.at[pl.ds(0, _EPW)],
                    tgt_out.at[pl.ds(wid * _EPW, _EPW)])


_sc_gather = functools.partial(
    pl.kernel,
    mesh=plsc.VectorSubcoreMesh(core_axis_name="c", subcore_axis_name="s"),
    out_type=[
        jax.ShapeDtypeStruct((_B * _W, _D), jnp.float32),
        jax.ShapeDtypeStruct((_B, _D), jnp.float32),
        jax.ShapeDtypeStruct((_B * _N, _D), jnp.float32),
    ],
    scratch_types=[
        pltpu.VMEM((_NCH, _CHUNK), jnp.int32),
        pltpu.VMEM((_ROWS_PW, _D), jnp.float32),
        pltpu.SemaphoreType.DMA,
    ],
    compiler_params=pltpu.CompilerParams(use_tc_tiling_on_sc=False),
)(_sc_gather_body)


_BB = 2048


def _tc_loss_body(ctx_ref, tgt_ref, neg_ref, out_ref):
    ctx = ctx_ref[...]
    tgt = tgt_ref[...]
    neg = neg_ref[...]

    m = ctx[:, 0:_D]
    for w in range(1, _W):
        m = m + ctx[:, w * _D:(w + 1) * _D]
    m = m * (1.0 / _W)

    def _nrm(x):
        sq = jnp.sum(x * x, axis=-1, keepdims=True)
        return x * lax.rsqrt(jnp.maximum(sq, 1e-12))

    mn = _nrm(m)
    tn = _nrm(tgt)
    pos_cos = jnp.sum(tn * mn, axis=-1, keepdims=True)
    part = jnp.sum(jnp.log(1.0 + jnp.exp(-pos_cos))) * (1.0 / _B)
    for j in range(_N):
        njn = _nrm(neg[:, j * _D:(j + 1) * _D])
        ncos = jnp.sum(tn * njn, axis=-1, keepdims=True)
        part = part + jnp.sum(jnp.log(1.0 + jnp.exp(ncos))) * (1.0 / (_B * _N))

    @pl.when(pl.program_id(0) == 0)
    def _():
        out_ref[...] = jnp.zeros_like(out_ref)

    out_ref[...] += jnp.full((1, 1), part, jnp.float32)


_tc_loss = pl.pallas_call(
    _tc_loss_body,
    grid=(_B // _BB,),
    in_specs=[
        pl.BlockSpec((_BB, _W * _D), lambda i: (i, 0)),
        pl.BlockSpec((_BB, _D), lambda i: (i, 0)),
        pl.BlockSpec((_BB, _N * _D), lambda i: (i, 0)),
    ],
    out_specs=pl.BlockSpec((1, 1), lambda i: (0, 0)),
    out_shape=jax.ShapeDtypeStruct((1, 1), jnp.float32),
)


def kernel(input_table, output_table, contexts, target, negatives):
    ctx_idx = contexts.astype(jnp.int32).reshape(_NW, _NCH, _CHUNK)
    neg_idx = negatives.astype(jnp.int32).reshape(_NW, _NCH, _CHUNK)
    tgt_idx = target.astype(jnp.int32).reshape(_NW, _NCH_T, _CHUNK)
    ctx_rows, tgt_rows, neg_rows = _sc_gather(
        input_table, output_table, ctx_idx, tgt_idx, neg_idx)
    loss = _tc_loss(ctx_rows.reshape(_B, _W * _D), tgt_rows,
                    neg_rows.reshape(_B, _N * _D))
    return loss[0, 0]


# speedup vs baseline: 1.0290x; 1.0290x over previous
import functools

import jax
import jax.numpy as jnp
from jax import lax
from jax.experimental import pallas as pl
from jax.experimental.pallas import tpu as pltpu
from jax.experimental.pallas import tpu_sc as plsc

_B = 16384
_D = 32
_W = 5
_N = 5
_V = 1000000

_NC = 2
_NS = 16
_NW = _NC * _NS
_EPW = _B // _NW
_CHUNK = 128
_ROWS_PW = _EPW * _W
_NCH = _ROWS_PW // _CHUNK
_NCH_T = _EPW // _CHUNK


_TB = 2048
_TSUB = _TB // 4
_NTB = pl.cdiv(_V, _TB)
_VP = _NTB * _TB


def _tc_transpose_body(in_ref, out_ref):
    x = in_ref[...]
    parts = [jnp.transpose(x[:, k * _TSUB:(k + 1) * _TSUB]) for k in range(4)]
    out_ref[...] = jnp.concatenate(parts, axis=1)


_tc_transpose = pl.pallas_call(
    _tc_transpose_body,
    grid=(_NTB,),
    in_specs=[pl.BlockSpec((_D, _TB), lambda i: (0, i))],
    out_specs=pl.BlockSpec((_TSUB, 128), lambda i: (i, 0)),
    out_shape=jax.ShapeDtypeStruct((_VP // 4, 128), jnp.float32),
)


def _permute_idx(idx_ref, nrows):
    for j in range(nrows):
        for h in range(8):
            v = idx_ref[j, pl.ds(h * 16, 16)]
            p = ((v & -2048) + ((v & 511) << 2)) + ((v >> 9) & 3)
            idx_ref[j, pl.ds(h * 16, 16)] = p


def _sc_gather_body(in_tab, out_tab, ctx_idx, tgt_idx, neg_idx,
                    ctx_out, tgt_out, neg_out, idx_v, rows_v, sem):
    c = lax.axis_index("c")
    s = lax.axis_index("s")
    wid = s * _NC + c

    pltpu.sync_copy(ctx_idx.at[wid], idx_v)
    _permute_idx(idx_v, _NCH)
    cps = [pltpu.async_copy(in_tab.at[idx_v.at[j]],
                            rows_v.at[pl.ds(j * _CHUNK, _CHUNK)], sem)
           for j in range(_NCH)]
    for cp in cps:
        cp.wait()
    pltpu.sync_copy(rows_v, ctx_out.at[pl.ds(wid * _ROWS_PW, _ROWS_PW)])

    pltpu.sync_copy(neg_idx.at[wid], idx_v)
    _permute_idx(idx_v, _NCH)
    cps = [pltpu.async_copy(out_tab.at[idx_v.at[j]],
                            rows_v.at[pl.ds(j * _CHUNK, _CHUNK)], sem)
           for j in range(_NCH)]
    for cp in cps:
        cp.wait()
    pltpu.sync_copy(rows_v, neg_out.at[pl.ds(wid * _ROWS_PW, _ROWS_PW)])

    pltpu.sync_copy(tgt_idx.at[wid], idx_v.at[pl.ds(0, _NCH_T)])
    _permute_idx(idx_v, _NCH_T)
    cps = [pltpu.async_copy(out_tab.at[idx_v.at[j]],
                            rows_v.at[pl.ds(j * _CHUNK, _CHUNK)], sem)
           for j in range(_NCH_T)]
    for cp in cps:
        cp.wait()
    pltpu.sync_copy(rows_v.at[pl.ds(0, _EPW)],
                    tgt_out.at[pl.ds(wid * _EPW, _EPW)])


_sc_gather = functools.partial(
    pl.kernel,
    mesh=plsc.VectorSubcoreMesh(core_axis_name="c", subcore_axis_name="s"),
    out_type=[
        jax.ShapeDtypeStruct((_B * _W, _D), jnp.float32),
        jax.ShapeDtypeStruct((_B, _D), jnp.float32),
        jax.ShapeDtypeStruct((_B * _N, _D), jnp.float32),
    ],
    scratch_types=[
        pltpu.VMEM((_NCH, _CHUNK), jnp.int32),
        pltpu.VMEM((_ROWS_PW, _D), jnp.float32),
        pltpu.SemaphoreType.DMA,
    ],
    compiler_params=pltpu.CompilerParams(use_tc_tiling_on_sc=False),
)(_sc_gather_body)


_BB = 2048


def _tc_loss_body(ctx_ref, tgt_ref, neg_ref, out_ref):
    ctx = ctx_ref[...]
    tgt = tgt_ref[...]
    neg = neg_ref[...]

    m = ctx[:, 0:_D]
    for w in range(1, _W):
        m = m + ctx[:, w * _D:(w + 1) * _D]
    m = m * (1.0 / _W)

    def _nrm(x):
        sq = jnp.sum(x * x, axis=-1, keepdims=True)
        return x * lax.rsqrt(jnp.maximum(sq, 1e-12))

    mn = _nrm(m)
    tn = _nrm(tgt)
    pos_cos = jnp.sum(tn * mn, axis=-1, keepdims=True)
    part = jnp.sum(jnp.log(1.0 + jnp.exp(-pos_cos))) * (1.0 / _B)
    for j in range(_N):
        njn = _nrm(neg[:, j * _D:(j + 1) * _D])
        ncos = jnp.sum(tn * njn, axis=-1, keepdims=True)
        part = part + jnp.sum(jnp.log(1.0 + jnp.exp(ncos))) * (1.0 / (_B * _N))

    @pl.when(pl.program_id(0) == 0)
    def _():
        out_ref[...] = jnp.zeros_like(out_ref)

    out_ref[...] += jnp.full((1, 1), part, jnp.float32)


_tc_loss = pl.pallas_call(
    _tc_loss_body,
    grid=(_B // _BB,),
    in_specs=[
        pl.BlockSpec((_BB, _W * _D), lambda i: (i, 0)),
        pl.BlockSpec((_BB, _D), lambda i: (i, 0)),
        pl.BlockSpec((_BB, _N * _D), lambda i: (i, 0)),
    ],
    out_specs=pl.BlockSpec((1, 1), lambda i: (0, 0)),
    out_shape=jax.ShapeDtypeStruct((1, 1), jnp.float32),
)


def kernel(input_table, output_table, contexts, target, negatives):
    in_lin = _tc_transpose(input_table.T).reshape(_VP, _D)
    out_lin = _tc_transpose(output_table.T).reshape(_VP, _D)
    ctx_idx = contexts.astype(jnp.int32).reshape(_NW, _NCH, _CHUNK)
    neg_idx = negatives.astype(jnp.int32).reshape(_NW, _NCH, _CHUNK)
    tgt_idx = target.astype(jnp.int32).reshape(_NW, _NCH_T, _CHUNK)
    ctx_rows, tgt_rows, neg_rows = _sc_gather(
        in_lin, out_lin, ctx_idx, tgt_idx, neg_idx)
    loss = _tc_loss(ctx_rows.reshape(_B, _W * _D), tgt_rows,
                    neg_rows.reshape(_B, _N * _D))
    return loss[0, 0]


# speedup vs baseline: 1.7182x; 1.6698x over previous
import functools

import jax
import jax.numpy as jnp
from jax import lax
from jax.experimental import pallas as pl
from jax.experimental.pallas import tpu as pltpu
from jax.experimental.pallas import tpu_sc as plsc

_B = 16384
_D = 32
_W = 5
_N = 5
_V = 1000000

_NC = 2
_NS = 16
_NW = _NC * _NS
_EPW = _B // _NW
_CHUNK = 128
_ROWS_PW = _EPW * _W
_NCH = _ROWS_PW // _CHUNK
_NCH_T = _EPW // _CHUNK


_TB = 8192
_TSUB = _TB // 4
_KSH = 11
_NTB = pl.cdiv(_V, _TB)
_VP = _NTB * _TB


_VTAIL = _V - (_NTB - 1) * _TB


def _tc_transpose_body(in_ref, out_ref):
    x = in_ref[...]
    x = lax.cond(
        pl.program_id(0) == _NTB - 1,
        lambda: jnp.where(
            lax.broadcasted_iota(jnp.int32, (_D, _TB), 1) < _VTAIL, x,
            jnp.float32(0.0)),
        lambda: x)
    sub = lax.broadcasted_iota(jnp.int32, (_D, 128), 0)
    lane = lax.broadcasted_iota(jnp.int32, (_D, 128), 1)
    acc = jnp.zeros((_TSUB, 128), jnp.float32)
    for k in range(4):
        ek = jnp.where(lane == 32 * k + sub, 1.0, 0.0)
        acc = acc + lax.dot_general(
            x[:, k * _TSUB:(k + 1) * _TSUB], ek,
            (((0,), (0,)), ((), ())), preferred_element_type=jnp.float32)
    out_ref[...] = acc


_tc_transpose = pl.pallas_call(
    _tc_transpose_body,
    grid=(_NTB,),
    in_specs=[pl.BlockSpec((_D, _TB), lambda i: (0, i))],
    out_specs=pl.BlockSpec((_TSUB, 128), lambda i: (i, 0)),
    out_shape=jax.ShapeDtypeStruct((_VP // 4, 128), jnp.float32),
)


def _permute_idx(idx_ref, nrows):
    for j in range(nrows):
        for h in range(8):
            v = idx_ref[j, pl.ds(h * 16, 16)]
            p = ((v & -_TB) + ((v & (_TSUB - 1)) << 2)) + ((v >> _KSH) & 3)
            idx_ref[j, pl.ds(h * 16, 16)] = p


def _sc_gather_body(in_tab, out_tab, ctx_idx, tgt_idx, neg_idx,
                    ctx_out, tgt_out, neg_out, idx_v, rows_v, sem):
    c = lax.axis_index("c")
    s = lax.axis_index("s")
    wid = s * _NC + c

    pltpu.sync_copy(ctx_idx.at[wid], idx_v)
    _permute_idx(idx_v, _NCH)
    cps = [pltpu.async_copy(in_tab.at[idx_v.at[j]],
                            rows_v.at[pl.ds(j * _CHUNK, _CHUNK)], sem)
           for j in range(_NCH)]
    for cp in cps:
        cp.wait()
    pltpu.sync_copy(rows_v, ctx_out.at[pl.ds(wid * _ROWS_PW, _ROWS_PW)])

    pltpu.sync_copy(neg_idx.at[wid], idx_v)
    _permute_idx(idx_v, _NCH)
    cps = [pltpu.async_copy(out_tab.at[idx_v.at[j]],
                            rows_v.at[pl.ds(j * _CHUNK, _CHUNK)], sem)
           for j in range(_NCH)]
    for cp in cps:
        cp.wait()
    pltpu.sync_copy(rows_v, neg_out.at[pl.ds(wid * _ROWS_PW, _ROWS_PW)])

    pltpu.sync_copy(tgt_idx.at[wid], idx_v.at[pl.ds(0, _NCH_T)])
    _permute_idx(idx_v, _NCH_T)
    cps = [pltpu.async_copy(out_tab.at[idx_v.at[j]],
                            rows_v.at[pl.ds(j * _CHUNK, _CHUNK)], sem)
           for j in range(_NCH_T)]
    for cp in cps:
        cp.wait()
    pltpu.sync_copy(rows_v.at[pl.ds(0, _EPW)],
                    tgt_out.at[pl.ds(wid * _EPW, _EPW)])


_sc_gather = functools.partial(
    pl.kernel,
    mesh=plsc.VectorSubcoreMesh(core_axis_name="c", subcore_axis_name="s"),
    out_type=[
        jax.ShapeDtypeStruct((_B * _W, _D), jnp.float32),
        jax.ShapeDtypeStruct((_B, _D), jnp.float32),
        jax.ShapeDtypeStruct((_B * _N, _D), jnp.float32),
    ],
    scratch_types=[
        pltpu.VMEM((_NCH, _CHUNK), jnp.int32),
        pltpu.VMEM((_ROWS_PW, _D), jnp.float32),
        pltpu.SemaphoreType.DMA,
    ],
    compiler_params=pltpu.CompilerParams(use_tc_tiling_on_sc=False),
)(_sc_gather_body)


_RB = 512


def _tc_loss_body(ctx_ref, tgt_ref, neg_ref, out_ref):
    ctx = ctx_ref[...]
    tgt = tgt_ref[...]
    neg = neg_ref[...]

    def _nrm(x):
        sq = jnp.sum(x * x, axis=-1, keepdims=True)
        return x * lax.rsqrt(jnp.maximum(sq, 1e-12))

    part = jnp.float32(0.0)
    for m in range(4):
        cm = ctx[:, m * _W * _D:(m + 1) * _W * _D]
        tm = tgt[:, m * _D:(m + 1) * _D]
        nm = neg[:, m * _W * _D:(m + 1) * _W * _D]
        s = cm[:, 0:_D]
        for w in range(1, _W):
            s = s + cm[:, w * _D:(w + 1) * _D]
        mn = _nrm(s * (1.0 / _W))
        tn = _nrm(tm)
        pos_cos = jnp.sum(tn * mn, axis=-1, keepdims=True)
        part = part + jnp.sum(jnp.log(1.0 + jnp.exp(-pos_cos))) * (1.0 / _B)
        for j in range(_N):
            njn = _nrm(nm[:, j * _D:(j + 1) * _D])
            ncos = jnp.sum(tn * njn, axis=-1, keepdims=True)
            part = part + (jnp.sum(jnp.log(1.0 + jnp.exp(ncos)))
                           * (1.0 / (_B * _N)))

    @pl.when(pl.program_id(0) == 0)
    def _():
        out_ref[...] = jnp.zeros_like(out_ref)

    out_ref[...] += jnp.full((1, 1), part, jnp.float32)


_tc_loss = pl.pallas_call(
    _tc_loss_body,
    grid=(_B // (4 * _RB),),
    in_specs=[
        pl.BlockSpec((_RB, 4 * _W * _D), lambda i: (i, 0)),
        pl.BlockSpec((_RB, 4 * _D), lambda i: (i, 0)),
        pl.BlockSpec((_RB, 4 * _N * _D), lambda i: (i, 0)),
    ],
    out_specs=pl.BlockSpec((1, 1), lambda i: (0, 0)),
    out_shape=jax.ShapeDtypeStruct((1, 1), jnp.float32),
)


def kernel(input_table, output_table, contexts, target, negatives):
    in_lin = _tc_transpose(input_table.T).reshape(_VP, _D)
    out_lin = _tc_transpose(output_table.T).reshape(_VP, _D)
    ctx_idx = contexts.astype(jnp.int32).reshape(_NW, _NCH, _CHUNK)
    neg_idx = negatives.astype(jnp.int32).reshape(_NW, _NCH, _CHUNK)
    tgt_idx = target.astype(jnp.int32).reshape(_NW, _NCH_T, _CHUNK)
    ctx_rows, tgt_rows, neg_rows = _sc_gather(
        in_lin, out_lin, ctx_idx, tgt_idx, neg_idx)
    loss = _tc_loss(ctx_rows.reshape(_B // 4, 4 * _W * _D),
                    tgt_rows.reshape(_B // 4, 4 * _D),
                    neg_rows.reshape(_B // 4, 4 * _N * _D))
    return loss[0, 0]
